# baseline (speedup 1.0000x reference)
import functools

import jax
import jax.numpy as jnp
from jax import lax
from jax.experimental import pallas as pl
from jax.experimental.pallas import tpu as pltpu
from jax.experimental.pallas import tpu_sc as plsc

N = 10000
D = 128
E = 320000

NC = 2
NS = 16
NW = NC * NS
L = 16

EP = 331776
EPW = EP // NW
CHUNK = 128

NPAD = 10240
NROWS = NPAD // D
RPT = NPAD // NS

RANGE = NPAD // NW
RROWS = EP // CHUNK
SCAN = 2048
FLUSH = 8192
SCAP = 11392
BLK = 8

_MESH = plsc.VectorSubcoreMesh(core_axis_name="c", subcore_axis_name="s")
_SC_PARAMS = pltpu.CompilerParams(needs_layout_passes=False)


def _zero_rows(ref, nrows):
    def body(i, carry):
        for j in range(D // L):
            ref[i, pl.ds(j * L, L)] = jnp.zeros((L,), jnp.float32)
        return carry
    lax.fori_loop(0, nrows, body, 0)


@functools.partial(
    pl.kernel,
    out_type=jax.ShapeDtypeStruct((NC * NPAD,), jnp.float32),
    mesh=_MESH,
    scratch_types=[
        pltpu.VMEM((EPW,), jnp.int32),
        pltpu.VMEM((EPW,), jnp.float32),
        pltpu.VMEM((NPAD,), jnp.float32),
        pltpu.VMEM((RPT,), jnp.float32),
        pltpu.VMEM((RPT,), jnp.float32),
        pltpu.VMEM_SHARED((NS, NPAD), jnp.float32),
    ],
    compiler_params=_SC_PARAMS,
)
def _deg_kernel(dste, ewe, out, dst_v, ew_v, hist, acc_v, buf_v, slabs):
    c = lax.axis_index("c")
    s = lax.axis_index("s")
    wid = s * NC + c
    pltpu.sync_copy(dste.at[pl.ds(wid * EPW, EPW)], dst_v)
    pltpu.sync_copy(ewe.at[pl.ds(wid * EPW, EPW)], ew_v)

    def zero(i, carry):
        hist[pl.ds(i * L, L)] = jnp.zeros((L,), jnp.float32)
        return carry
    lax.fori_loop(0, NPAD // L, zero, 0)

    def acc(i, carry):
        dvec = dst_v[pl.ds(i * L, L)]
        wvec = ew_v[pl.ds(i * L, L)]
        plsc.addupdate_scatter(hist, [dvec], wvec)
        return carry
    lax.fori_loop(0, EPW // L, acc, 0)

    pltpu.sync_copy(hist, slabs.at[s])
    plsc.subcore_barrier()

    base = s * RPT

    def zacc(i, carry):
        acc_v[pl.ds(i * L, L)] = jnp.zeros((L,), jnp.float32)
        return carry
    lax.fori_loop(0, RPT // L, zacc, 0)
    for t in range(NS):
        pltpu.sync_copy(slabs.at[t, pl.ds(base, RPT)], buf_v)

        def radd(i, carry):
            sl = pl.ds(i * L, L)
            acc_v[sl] = acc_v[sl] + buf_v[sl]
            return carry
        lax.fori_loop(0, RPT // L, radd, 0)
    pltpu.sync_copy(acc_v, out.at[pl.ds(c * NPAD + base, RPT)])


@functools.partial(
    pl.kernel,
    out_type=(
        jax.ShapeDtypeStruct((NW * EP,), jnp.int32),
        jax.ShapeDtypeStruct((NW * EP,), jnp.int32),
        jax.ShapeDtypeStruct((NW * EP,), jnp.float32),
        jax.ShapeDtypeStruct((NW * L,), jnp.int32),
    ),
    mesh=_MESH,
    scratch_types=[
        pltpu.VMEM((SCAN,), jnp.int32),
        pltpu.VMEM((SCAN,), jnp.int32),
        pltpu.VMEM((SCAN,), jnp.float32),
        pltpu.VMEM((SCAP,), jnp.int32),
        pltpu.VMEM((SCAP,), jnp.int32),
        pltpu.VMEM((SCAP,), jnp.float32),
        pltpu.VMEM((L,), jnp.int32),
    ],
    compiler_params=_SC_PARAMS,
)
def _route_kernel(srce, dste, ewe, rsrc, rdst, rew, rcnt,
                  sb_v, db_v, wb_v, st_s, st_d, st_w, cnt_v):
    c = lax.axis_index("c")
    s = lax.axis_index("s")
    wid = s * NC + c
    r0 = wid * RANGE

    hb0 = wid * EP

    def flush_rows(hbrow, nrows):
        def frow(r, carry):
            pltpu.sync_copy(st_s.at[pl.ds(r * CHUNK, CHUNK)],
                            rsrc.at[pl.ds(hb0 + (hbrow + r) * CHUNK, CHUNK)])
            pltpu.sync_copy(st_d.at[pl.ds(r * CHUNK, CHUNK)],
                            rdst.at[pl.ds(hb0 + (hbrow + r) * CHUNK, CHUNK)])
            pltpu.sync_copy(st_w.at[pl.ds(r * CHUNK, CHUNK)],
                            rew.at[pl.ds(hb0 + (hbrow + r) * CHUNK, CHUNK)])
            return carry
        lax.fori_loop(0, nrows, frow, 0)

    def batch(bi, carry):
        cur, hbrow = carry
        eb = bi * SCAN
        pltpu.sync_copy(srce.at[pl.ds(eb, SCAN)], sb_v)
        pltpu.sync_copy(dste.at[pl.ds(eb, SCAN)], db_v)
        pltpu.sync_copy(ewe.at[pl.ds(eb, SCAN)], wb_v)

        def group(i, cur_):
            sl = pl.ds(i * L, L)
            dvec = db_v[sl]
            m = (dvec >= r0) & (dvec < r0 + RANGE)
            plsc.store_compressed(st_s.at[pl.ds(cur_, L)], sb_v[sl], mask=m)
            plsc.store_compressed(st_d.at[pl.ds(cur_, L)], dvec, mask=m)
            plsc.store_compressed(st_w.at[pl.ds(cur_, L)], wb_v[sl], mask=m)
            npick = plsc.all_reduce_population_count(m)
            return cur_ + npick[0]
        cur = lax.fori_loop(0, SCAN // L, group, cur)

        do_flush = cur >= FLUSH

        @pl.when(do_flush)
        def _():
            flush_rows(hbrow, FLUSH // CHUNK)
            def mv(i, carry):
                sl_hi = pl.ds(FLUSH + i * L, L)
                sl_lo = pl.ds(i * L, L)
                st_s[sl_lo] = st_s[sl_hi]
                st_d[sl_lo] = st_d[sl_hi]
                st_w[sl_lo] = st_w[sl_hi]
                return carry
            lax.fori_loop(0, (SCAN + 256) // L, mv, 0)

        cur = jnp.where(do_flush, cur - FLUSH, cur)
        hbrow = jnp.where(do_flush, hbrow + FLUSH // CHUNK, hbrow)
        return (cur, hbrow)

    cur, hbrow = lax.fori_loop(0, EP // SCAN, batch,
                               (jnp.int32(0), jnp.int32(0)))

    full = jnp.ones((L,), jnp.bool_)
    zi = jnp.zeros((L,), jnp.int32)
    zf = jnp.zeros((L,), jnp.float32)
    dsentinel = zi + r0
    for t in range(BLK * CHUNK // L):
        plsc.store_compressed(st_s.at[pl.ds(cur + t * L, L)], zi, mask=full)
        plsc.store_compressed(st_d.at[pl.ds(cur + t * L, L)], dsentinel, mask=full)
        plsc.store_compressed(st_w.at[pl.ds(cur + t * L, L)], zf, mask=full)
    curp = (cur + BLK * CHUNK - 1) & ~(BLK * CHUNK - 1)
    flush_rows(hbrow, lax.shift_right_logical(curp, 7))
    totrows = hbrow + lax.shift_right_logical(curp, 7)
    cnt_v[pl.ds(0, L)] = jnp.broadcast_to(totrows, (L,)).astype(jnp.int32)
    pltpu.sync_copy(cnt_v, rcnt.at[pl.ds(wid * L, L)])


@functools.partial(
    pl.kernel,
    out_type=jax.ShapeDtypeStruct((NPAD, D), jnp.float32),
    mesh=_MESH,
    scratch_types=[
        pltpu.VMEM((BLK * CHUNK,), jnp.int32),
        pltpu.VMEM((BLK * CHUNK,), jnp.int32),
        pltpu.VMEM((BLK * CHUNK,), jnp.float32),
        pltpu.VMEM((NPAD,), jnp.float32),
        pltpu.VMEM((L,), jnp.int32),
        pltpu.VMEM((CHUNK, D), jnp.float32),
        pltpu.VMEM((CHUNK, D), jnp.float32),
        pltpu.VMEM((RANGE, D), jnp.float32),
        pltpu.SemaphoreType.DMA,
    ],
    compiler_params=_SC_PARAMS,
)
def _edge_kernel(table, rsrc, rdst, rew, rcnt, dis, out,
                 src_v, dst_v, ew_v, dis_v, cnt_v, rows_a, rows_b, accum,
                 gsem):
    c = lax.axis_index("c")
    s = lax.axis_index("s")
    wid = s * NC + c
    r0 = wid * RANGE
    hb0 = wid * EP
    bufs = (rows_a, rows_b)

    pltpu.sync_copy(dis, dis_v)
    pltpu.sync_copy(rcnt.at[pl.ds(wid * L, L)], cnt_v)
    _zero_rows(accum, RANGE)
    nrows = cnt_v[pl.ds(0, L)][0]
    nblk = lax.shift_right_logical(nrows + (BLK - 1), 3)

    def proc_chunk(rows_v, g):
        def s16(i, cc):
            sl = pl.ds(g * CHUNK + i * L, L)
            svec = src_v[sl]
            dvec = dst_v[sl]
            wvec = ew_v[sl]
            nv = plsc.load_gather(dis_v, [svec]) * wvec * plsc.load_gather(dis_v, [dvec])
            dloc = dvec - r0
            e0 = i * L
            for e in range(L):
                w = nv[e]
                dl = dloc[e]
                for j in range(D // L):
                    plsc.addupdate(accum.at[dl, pl.ds(j * L, L)],
                                   rows_v[e0 + e, pl.ds(j * L, L)] * w)
            return cc
        lax.fori_loop(0, CHUNK // L, s16, 0)

    def block(b, carry):
        base = hb0 + b * (BLK * CHUNK)
        pltpu.sync_copy(rsrc.at[pl.ds(base, BLK * CHUNK)], src_v)
        pltpu.sync_copy(rdst.at[pl.ds(base, BLK * CHUNK)], dst_v)
        pltpu.sync_copy(rew.at[pl.ds(base, BLK * CHUNK)], ew_v)
        gathers = [None] * BLK
        gathers[0] = pltpu.async_copy(
            table.at[src_v.at[pl.ds(0, CHUNK)]], bufs[0], gsem)
        for g in range(BLK):
            gathers[g].wait()
            if g + 1 < BLK:
                gathers[g + 1] = pltpu.async_copy(
                    table.at[src_v.at[pl.ds((g + 1) * CHUNK, CHUNK)]],
                    bufs[(g + 1) % 2], gsem)
            proc_chunk(bufs[g % 2], g)
        return carry
    lax.fori_loop(0, nblk, block, 0)

    pltpu.sync_copy(accum, out.at[pl.ds(r0, RANGE)])


RB = 512


def _dis_body(dp_ref, o_ref):
    dp = dp_ref[...]
    deg = dp[:NROWS, :] + dp[NROWS:, :]
    o_ref[...] = jnp.where(deg > 0, lax.rsqrt(jnp.maximum(deg, 1e-12)), 0.0)


def _mm_body(x_ref, w_ref, o_ref):
    o_ref[...] = jnp.dot(x_ref[...], w_ref[...], preferred_element_type=jnp.float32)


def _mid_body(s_ref, b_ref, w_ref, o_ref):
    a = jnp.maximum(s_ref[...] + b_ref[...], 0.0)
    o_ref[...] = jnp.dot(a, w_ref[...], preferred_element_type=jnp.float32)


def _fin_body(s_ref, b_ref, o_ref):
    o_ref[...] = s_ref[...] + b_ref[...]


def _tc_dis(degp):
    return pl.pallas_call(
        _dis_body,
        out_shape=jax.ShapeDtypeStruct((NROWS, D), jnp.float32),
    )(degp)


def _tc_mm(x, w):
    return pl.pallas_call(
        _mm_body,
        grid=(NPAD // RB,),
        in_specs=[pl.BlockSpec((RB, D), lambda i: (i, 0)),
                  pl.BlockSpec((D, D), lambda i: (0, 0))],
        out_specs=pl.BlockSpec((RB, D), lambda i: (i, 0)),
        out_shape=jax.ShapeDtypeStruct((NPAD, D), jnp.float32),
    )(x, w)


def _tc_mid(sarr, b, w):
    return pl.pallas_call(
        _mid_body,
        grid=(NPAD // RB,),
        in_specs=[pl.BlockSpec((RB, D), lambda i: (i, 0)),
                  pl.BlockSpec((1, D), lambda i: (0, 0)),
                  pl.BlockSpec((D, D), lambda i: (0, 0))],
        out_specs=pl.BlockSpec((RB, D), lambda i: (i, 0)),
        out_shape=jax.ShapeDtypeStruct((NPAD, D), jnp.float32),
    )(sarr, b, w)


def _tc_fin(sarr, b):
    return pl.pallas_call(
        _fin_body,
        grid=(NPAD // RB,),
        in_specs=[pl.BlockSpec((RB, D), lambda i: (i, 0)),
                  pl.BlockSpec((1, D), lambda i: (0, 0))],
        out_specs=pl.BlockSpec((RB, D), lambda i: (i, 0)),
        out_shape=jax.ShapeDtypeStruct((NPAD, D), jnp.float32),
    )(sarr, b)


def kernel(x, edge_index, edge_attr, W1, b1, W2, b2):
    src = edge_index[0]
    dst = edge_index[1]
    loop = jnp.arange(N, dtype=jnp.int32)
    padn = EP - E - N
    srce = jnp.concatenate([src, loop, jnp.zeros((padn,), jnp.int32)])
    dste = jnp.concatenate([dst, loop, jnp.zeros((padn,), jnp.int32)])
    ewe = jnp.concatenate([edge_attr, jnp.ones((N,), jnp.float32),
                           jnp.zeros((padn,), jnp.float32)])
    xp = jnp.pad(x, ((0, NPAD - N), (0, 0)))

    degp = _deg_kernel(dste, ewe).reshape(NC * NROWS, D)
    dis = _tc_dis(degp).reshape(NPAD)
    rsrc, rdst, rew, rcnt = _route_kernel(srce, dste, ewe)
    h1 = _tc_mm(xp, W1)
    S1 = _edge_kernel(h1, rsrc, rdst, rew, rcnt, dis)
    h2 = _tc_mid(S1, b1.reshape(1, D), W2)
    S2 = _edge_kernel(h2, rsrc, rdst, rew, rcnt, dis)
    out = _tc_fin(S2, b2.reshape(1, D))
    return out[:N]


# speedup vs baseline: 3.3337x; 3.3337x over previous
import functools

import jax
import jax.numpy as jnp
import numpy as np
from jax import lax
from jax.experimental import pallas as pl
from jax.experimental.pallas import tpu as pltpu
from jax.experimental.pallas import tpu_sc as plsc

N = 10000
D = 128
E = 320000

NC = 2
NS = 16
NW = NC * NS
L = 16

CHUNK = 128
EPW = 10496
NCH = EPW // CHUNK
SUP = 2
NSUP = NCH // SUP
EP = NW * EPW

NPAD = 10240
NROWS = NPAD // D
RPT = NPAD // NS

_MESH = plsc.VectorSubcoreMesh(core_axis_name="c", subcore_axis_name="s")
_SC_PARAMS = pltpu.CompilerParams(needs_layout_passes=False)


def _zero_rows(ref, nrows):
    def body(i, carry):
        for j in range(D // L):
            ref[i, pl.ds(j * L, L)] = jnp.zeros((L,), jnp.float32)
        return carry
    lax.fori_loop(0, nrows, body, 0)


@functools.partial(
    pl.kernel,
    out_type=jax.ShapeDtypeStruct((NC * NPAD,), jnp.float32),
    mesh=_MESH,
    scratch_types=[
        pltpu.VMEM((EPW,), jnp.int32),
        pltpu.VMEM((EPW,), jnp.float32),
        pltpu.VMEM((NPAD,), jnp.float32),
        pltpu.VMEM((RPT,), jnp.float32),
        pltpu.VMEM((RPT,), jnp.float32),
        pltpu.VMEM_SHARED((NS, NPAD), jnp.float32),
    ],
    compiler_params=_SC_PARAMS,
)
def _deg_kernel(dste, ewe, out, dst_v, ew_v, hist, acc_v, buf_v, slabs):
    c = lax.axis_index("c")
    s = lax.axis_index("s")
    wid = s * NC + c
    pltpu.sync_copy(dste.at[pl.ds(wid * EPW, EPW)], dst_v)
    pltpu.sync_copy(ewe.at[pl.ds(wid * EPW, EPW)], ew_v)

    def zero(i, carry):
        hist[pl.ds(i * L, L)] = jnp.zeros((L,), jnp.float32)
        return carry
    lax.fori_loop(0, NPAD // L, zero, 0)

    def acc(i, carry):
        dvec = dst_v[pl.ds(i * L, L)]
        wvec = ew_v[pl.ds(i * L, L)]
        plsc.addupdate_scatter(hist, [dvec], wvec)
        return carry
    lax.fori_loop(0, EPW // L, acc, 0)

    pltpu.sync_copy(hist, slabs.at[s])
    plsc.subcore_barrier()

    base = s * RPT

    def zacc(i, carry):
        acc_v[pl.ds(i * L, L)] = jnp.zeros((L,), jnp.float32)
        return carry
    lax.fori_loop(0, RPT // L, zacc, 0)
    for t in range(NS):
        pltpu.sync_copy(slabs.at[t, pl.ds(base, RPT)], buf_v)

        def radd(i, carry):
            sl = pl.ds(i * L, L)
            acc_v[sl] = acc_v[sl] + buf_v[sl]
            return carry
        lax.fori_loop(0, RPT // L, radd, 0)
    pltpu.sync_copy(acc_v, out.at[pl.ds(c * NPAD + base, RPT)])


@functools.partial(
    pl.kernel,
    out_type=jax.ShapeDtypeStruct((NC * NPAD, D), jnp.float32),
    mesh=_MESH,
    scratch_types=[
        pltpu.VMEM((SUP * CHUNK,), jnp.int32),
        pltpu.VMEM((SUP, CHUNK), jnp.int32),
        pltpu.VMEM((SUP * CHUNK,), jnp.float32),
        pltpu.VMEM((NPAD,), jnp.float32),
        pltpu.VMEM((CHUNK, D), jnp.float32),
        pltpu.VMEM((CHUNK, D), jnp.float32),
        pltpu.VMEM_SHARED((NPAD, D), jnp.float32),
        pltpu.SemaphoreType.DMA,
        pltpu.SemaphoreType.DMA,
    ],
    compiler_params=_SC_PARAMS,
)
def _edge_kernel(table, srce, dste, ewe, dis, out,
                 src_v, dst_v, ew_v, dis_v, rows_a, rows_b, accum, gsem, ssem):
    c = lax.axis_index("c")
    s = lax.axis_index("s")
    wid = s * NC + c
    bufs = (rows_a, rows_b)
    pltpu.sync_copy(dis, dis_v)

    _zero_rows(rows_a, CHUNK)
    for k in range(RPT // CHUNK):
        pltpu.sync_copy(rows_a, accum.at[pl.ds(s * RPT + k * CHUNK, CHUNK)])
    plsc.subcore_barrier()

    def scale_chunk(rows_v, g):
        eb = g * CHUNK

        def scale16(i, cc):
            svec = src_v[pl.ds(eb + i * L, L)]
            dvec = dst_v[g, pl.ds(i * L, L)]
            wvec = ew_v[pl.ds(eb + i * L, L)]
            nv = plsc.load_gather(dis_v, [svec]) * wvec * plsc.load_gather(dis_v, [dvec])
            e0 = i * L
            for e in range(L):
                w = nv[e]
                for j in range(D // L):
                    rows_v[e0 + e, pl.ds(j * L, L)] = rows_v[e0 + e, pl.ds(j * L, L)] * w
            return cc
        lax.fori_loop(0, CHUNK // L, scale16, 0)

    def sup_body(u, carry):
        eb0 = wid * EPW + u * (SUP * CHUNK)
        pltpu.sync_copy(srce.at[pl.ds(eb0, SUP * CHUNK)], src_v)
        pltpu.sync_copy(ewe.at[pl.ds(eb0, SUP * CHUNK)], ew_v)
        pltpu.sync_copy(dste.at[wid, u], dst_v)

        gathers = [None] * SUP
        scatters = [None] * SUP
        gathers[0] = pltpu.async_copy(
            table.at[src_v.at[pl.ds(0, CHUNK)]], bufs[0], gsem)
        for g in range(SUP):
            buf = bufs[g % 2]
            gathers[g].wait()
            if g >= 1:
                scatters[g - 1].wait()
            if g + 1 < SUP:
                gathers[g + 1] = pltpu.async_copy(
                    table.at[src_v.at[pl.ds((g + 1) * CHUNK, CHUNK)]],
                    bufs[(g + 1) % 2], gsem)
            scale_chunk(buf, g)
            scatters[g] = pltpu.async_copy(
                buf, accum.at[dst_v.at[g]], ssem, add=True)
        scatters[SUP - 1].wait()
        return carry
    lax.fori_loop(0, NSUP, sup_body, 0)

    plsc.subcore_barrier()
    for k in range(RPT // CHUNK):
        r0 = s * RPT + k * CHUNK
        pltpu.sync_copy(accum.at[pl.ds(r0, CHUNK)], rows_a)
        pltpu.sync_copy(rows_a, out.at[pl.ds(c * NPAD + r0, CHUNK)])


RB = 512


def _dis_body(dp_ref, o_ref):
    dp = dp_ref[...]
    deg = dp[:NROWS, :] + dp[NROWS:, :]
    o_ref[...] = jnp.where(deg > 0, lax.rsqrt(jnp.maximum(deg, 1e-12)), 0.0)


def _mm_body(x_ref, w_ref, o_ref):
    o_ref[...] = jnp.dot(x_ref[...], w_ref[...], preferred_element_type=jnp.float32)


def _mid_body(s0_ref, s1_ref, b_ref, w_ref, o_ref):
    a = jnp.maximum(s0_ref[...] + s1_ref[...] + b_ref[...], 0.0)
    o_ref[...] = jnp.dot(a, w_ref[...], preferred_element_type=jnp.float32)


def _fin_body(s0_ref, s1_ref, b_ref, o_ref):
    o_ref[...] = s0_ref[...] + s1_ref[...] + b_ref[...]


def _tc_dis(degp):
    return pl.pallas_call(
        _dis_body,
        out_shape=jax.ShapeDtypeStruct((NROWS, D), jnp.float32),
    )(degp)


def _tc_mm(x, w):
    return pl.pallas_call(
        _mm_body,
        grid=(NPAD // RB,),
        in_specs=[pl.BlockSpec((RB, D), lambda i: (i, 0)),
                  pl.BlockSpec((D, D), lambda i: (0, 0))],
        out_specs=pl.BlockSpec((RB, D), lambda i: (i, 0)),
        out_shape=jax.ShapeDtypeStruct((NPAD, D), jnp.float32),
    )(x, w)


def _tc_mid(s0, s1, b, w):
    return pl.pallas_call(
        _mid_body,
        grid=(NPAD // RB,),
        in_specs=[pl.BlockSpec((RB, D), lambda i: (i, 0)),
                  pl.BlockSpec((RB, D), lambda i: (i, 0)),
                  pl.BlockSpec((1, D), lambda i: (0, 0)),
                  pl.BlockSpec((D, D), lambda i: (0, 0))],
        out_specs=pl.BlockSpec((RB, D), lambda i: (i, 0)),
        out_shape=jax.ShapeDtypeStruct((NPAD, D), jnp.float32),
    )(s0, s1, b, w)


def _tc_fin(s0, s1, b):
    return pl.pallas_call(
        _fin_body,
        grid=(NPAD // RB,),
        in_specs=[pl.BlockSpec((RB, D), lambda i: (i, 0)),
                  pl.BlockSpec((RB, D), lambda i: (i, 0)),
                  pl.BlockSpec((1, D), lambda i: (0, 0))],
        out_specs=pl.BlockSpec((RB, D), lambda i: (i, 0)),
        out_shape=jax.ShapeDtypeStruct((NPAD, D), jnp.float32),
    )(s0, s1, b)


def kernel(x, edge_index, edge_attr, W1, b1, W2, b2):
    src = edge_index[0]
    dst = edge_index[1]
    loop = jnp.arange(N, dtype=jnp.int32)
    padn = EP - E - N
    srce = jnp.concatenate([src, loop, jnp.zeros((padn,), jnp.int32)])
    dste = jnp.concatenate([dst, loop, jnp.zeros((padn,), jnp.int32)])
    ewe = jnp.concatenate([edge_attr, jnp.ones((N,), jnp.float32),
                           jnp.zeros((padn,), jnp.float32)])
    dste3 = dste.reshape(NW, NSUP, SUP, CHUNK)
    xp = jnp.pad(x, ((0, NPAD - N), (0, 0)))

    degp = _deg_kernel(dste, ewe).reshape(NC * NROWS, D)
    dis = _tc_dis(degp).reshape(NPAD)
    h1 = _tc_mm(xp, W1)
    S1 = _edge_kernel(h1, srce, dste3, ewe, dis)
    h2 = _tc_mid(S1[:NPAD], S1[NPAD:], b1.reshape(1, D), W2)
    S2 = _edge_kernel(h2, srce, dste3, ewe, dis)
    out = _tc_fin(S2[:NPAD], S2[NPAD:], b2.reshape(1, D))
    return out[:N]
